# baseline (speedup 1.0000x reference)
import functools

import jax
import jax.numpy as jnp
from jax import lax
from jax.experimental import pallas as pl

N = 1024
D = 64
K = 16
BLK_I = 128
GRID = N // BLK_I
ROWS2 = BLK_I * K


def _knn_kernel(x_ref, xt_ref, idx_ref):
    i = pl.program_id(0)
    x_blk = x_ref[...]
    acc = jnp.zeros((BLK_I, N), jnp.float32)
    for g in range(D // 8):
        s = []
        for t in range(8 * g, 8 * g + 8):
            xi = x_blk[:, t:t + 1]
            xj = xt_ref[t:t + 1, :]
            df = xi - xj
            s.append(df * df)
        tree = ((s[0] + s[4]) + (s[2] + s[6])) + ((s[1] + s[5]) + (s[3] + s[7]))
        acc = acc + tree
    dist = jnp.sqrt(acc)
    jiota = lax.broadcasted_iota(jnp.int32, (BLK_I, N), 1)
    jiota_f = jiota.astype(jnp.float32)
    gid = i * BLK_I + lax.broadcasted_iota(jnp.int32, (BLK_I, N), 0)
    inf = jnp.float32(jnp.inf)
    dist = jnp.where(jiota == gid, inf, dist)
    big = jnp.float32(2.0 * N)
    cols = []
    for _ in range(K):
        m = jnp.min(dist, axis=1, keepdims=True)
        cand = jnp.where(dist == m, jiota_f, big)
        am = jnp.min(cand, axis=1, keepdims=True)
        cols.append(am.astype(jnp.int32))
        dist = jnp.where(jiota_f == am, inf, dist)
    idx_ref[...] = jnp.concatenate(cols, axis=1)


def _mlp_kernel(idx_ref, x_ref, w1t_ref, b1_ref, w2t_ref, b2_ref, out_ref):
    i = pl.program_id(0)
    idx_col = idx_ref[0]
    riota = lax.broadcasted_iota(jnp.int32, (ROWS2, 1), 0)
    self_col = i * BLK_I + (riota >> 4)
    piota = lax.broadcasted_iota(jnp.int32, (ROWS2, N), 1)
    m_pos = (piota == self_col).astype(jnp.float32)
    m_neg = (piota == idx_col).astype(jnp.float32)
    m = m_pos - m_neg
    v = jnp.dot(x_ref[...], w1t_ref[...], preferred_element_type=jnp.float32,
                precision=jax.lax.Precision.HIGHEST)
    m16 = m.astype(jnp.bfloat16)
    va = v.astype(jnp.bfloat16)
    vb = (v - va.astype(jnp.float32)).astype(jnp.bfloat16)
    hpre = (jnp.dot(m16, va, preferred_element_type=jnp.float32)
            + jnp.dot(m16, vb, preferred_element_type=jnp.float32))
    h = jnp.maximum(hpre + b1_ref[...], 0.0)
    out = (jnp.dot(h, w2t_ref[...], preferred_element_type=jnp.float32)
           + b2_ref[...])
    out_ref[0] = out


@functools.partial(jax.jit, static_argnames=("interpret",))
def _run(x, W1, b1, W2, b2, interpret=False):
    xm = x[0]
    xt = xm.T
    idx = pl.pallas_call(
        _knn_kernel,
        grid=(GRID,),
        in_specs=[
            pl.BlockSpec((BLK_I, D), lambda i: (i, 0)),
            pl.BlockSpec((D, N), lambda i: (0, 0)),
        ],
        out_specs=pl.BlockSpec((BLK_I, K), lambda i: (i, 0)),
        out_shape=jax.ShapeDtypeStruct((N, K), jnp.int32),
        interpret=interpret,
    )(xm, xt)

    idx3 = idx.reshape(GRID, ROWS2, 1)
    out = pl.pallas_call(
        _mlp_kernel,
        grid=(GRID,),
        in_specs=[
            pl.BlockSpec((1, ROWS2, 1), lambda i: (i, 0, 0)),
            pl.BlockSpec((N, D), lambda i: (0, 0)),
            pl.BlockSpec((D, D), lambda i: (0, 0)),
            pl.BlockSpec((1, D), lambda i: (0, 0)),
            pl.BlockSpec((D, D), lambda i: (0, 0)),
            pl.BlockSpec((1, D), lambda i: (0, 0)),
        ],
        out_specs=pl.BlockSpec((1, ROWS2, D), lambda i: (i, 0, 0)),
        out_shape=jax.ShapeDtypeStruct((GRID, ROWS2, D), jnp.float32),
        interpret=interpret,
    )(idx3, xm, W1.T, b1.reshape(1, D), W2.T, b2.reshape(1, D))
    return out.reshape(1, N, K, D)


def kernel(x, W1, b1, W2, b2, k):
    return _run(x, W1, b1, W2, b2)


# speedup vs baseline: 1.1951x; 1.1951x over previous
import functools

import jax
import jax.numpy as jnp
from jax import lax
from jax.experimental import pallas as pl

N = 1024
D = 64
K = 16
BLK_I = 128
GRID = N // BLK_I
ROWS2 = BLK_I * K


def _knn_kernel(x_ref, xt_ref, idx_ref):
    i = pl.program_id(0)
    x_blk = x_ref[...]
    acc = jnp.zeros((BLK_I, N), jnp.float32)
    for g in range(D // 8):
        s = []
        for t in range(8 * g, 8 * g + 8):
            xi = x_blk[:, t:t + 1]
            xj = xt_ref[t:t + 1, :]
            df = xi - xj
            s.append(df * df)
        tree = ((s[0] + s[4]) + (s[2] + s[6])) + ((s[1] + s[5]) + (s[3] + s[7]))
        acc = acc + tree
    dist = jnp.sqrt(acc)
    jiota = lax.broadcasted_iota(jnp.int32, (BLK_I, N), 1)
    jiota_f = jiota.astype(jnp.float32)
    gid = i * BLK_I + lax.broadcasted_iota(jnp.int32, (BLK_I, N), 0)
    inf = jnp.float32(jnp.inf)
    dist = jnp.where(jiota == gid, inf, dist)
    big = jnp.float32(2.0 * N)
    cols = []
    for _ in range(K):
        m = jnp.min(dist, axis=1, keepdims=True)
        cand = jnp.where(dist == m, jiota_f, big)
        am = jnp.min(cand, axis=1, keepdims=True)
        cols.append(am.astype(jnp.int32))
        dist = jnp.where(jiota_f == am, inf, dist)
    idx_ref[...] = jnp.concatenate(cols, axis=1)


def _mlp_kernel(idx_ref, x_ref, w1t_ref, b1_ref, w2t_ref, b2_ref, out_ref):
    i = pl.program_id(0)
    idx_blk = idx_ref[0]
    v = jnp.dot(x_ref[...], w1t_ref[...], preferred_element_type=jnp.float32,
                precision=jax.lax.Precision.HIGHEST)
    x_blk = x_ref[pl.ds(i * BLK_I, BLK_I), :]
    v_blk = jnp.dot(x_blk, w1t_ref[...], preferred_element_type=jnp.float32,
                    precision=jax.lax.Precision.HIGHEST)
    piota = lax.broadcasted_iota(jnp.int32, (BLK_I, N), 1)
    m_blocks = [(piota == idx_blk[:, s:s + 1]).astype(jnp.bfloat16)
                for s in range(K)]
    m_neg = jnp.concatenate(m_blocks, axis=0)
    v_self = jnp.concatenate([v_blk] * K, axis=0)
    va = v.astype(jnp.bfloat16)
    vn = jnp.dot(m_neg, va, preferred_element_type=jnp.float32)
    h = jnp.maximum((v_self - vn) + b1_ref[...], 0.0)
    out = (jnp.dot(h, w2t_ref[...], preferred_element_type=jnp.float32)
           + b2_ref[...])
    out_ref[0] = out.reshape(K, BLK_I, D)


@functools.partial(jax.jit, static_argnames=("interpret",))
def _run(x, W1, b1, W2, b2, interpret=False):
    xm = x[0]
    xt = xm.T
    idx = pl.pallas_call(
        _knn_kernel,
        grid=(GRID,),
        in_specs=[
            pl.BlockSpec((BLK_I, D), lambda i: (i, 0)),
            pl.BlockSpec((D, N), lambda i: (0, 0)),
        ],
        out_specs=pl.BlockSpec((BLK_I, K), lambda i: (i, 0)),
        out_shape=jax.ShapeDtypeStruct((N, K), jnp.int32),
        interpret=interpret,
    )(xm, xt)

    idx3 = idx.reshape(GRID, BLK_I, K)
    out = pl.pallas_call(
        _mlp_kernel,
        grid=(GRID,),
        in_specs=[
            pl.BlockSpec((1, BLK_I, K), lambda i: (i, 0, 0)),
            pl.BlockSpec((N, D), lambda i: (0, 0)),
            pl.BlockSpec((D, D), lambda i: (0, 0)),
            pl.BlockSpec((1, D), lambda i: (0, 0)),
            pl.BlockSpec((D, D), lambda i: (0, 0)),
            pl.BlockSpec((1, D), lambda i: (0, 0)),
        ],
        out_specs=pl.BlockSpec((1, K, BLK_I, D), lambda i: (i, 0, 0, 0)),
        out_shape=jax.ShapeDtypeStruct((GRID, K, BLK_I, D), jnp.float32),
        interpret=interpret,
    )(idx3, xm, W1.T, b1.reshape(1, D), W2.T, b2.reshape(1, D))
    return out.transpose(0, 2, 1, 3).reshape(1, N, K, D)


def kernel(x, W1, b1, W2, b2, k):
    return _run(x, W1, b1, W2, b2)


# speedup vs baseline: 1.3428x; 1.1236x over previous
import functools

import jax
import jax.numpy as jnp
from jax import lax
from jax.experimental import pallas as pl

N = 1024
D = 64
K = 16
BLK_I = 128
GRID = N // BLK_I
ROWS2 = BLK_I * K


def _fused_kernel(xf_ref, xt_ref, w1t_ref, b1_ref, w2t_ref, b2_ref, out_ref):
    i = pl.program_id(0)
    x_blk = xf_ref[pl.ds(i * BLK_I, BLK_I), :]
    acc = jnp.zeros((BLK_I, N), jnp.float32)
    for g in range(D // 8):
        s = []
        for t in range(8 * g, 8 * g + 8):
            xi = x_blk[:, t:t + 1]
            xj = xt_ref[t:t + 1, :]
            df = xi - xj
            s.append(df * df)
        tree = ((s[0] + s[4]) + (s[2] + s[6])) + ((s[1] + s[5]) + (s[3] + s[7]))
        acc = acc + tree
    dist = jnp.sqrt(acc)
    jiota = lax.broadcasted_iota(jnp.int32, (BLK_I, N), 1)
    jiota_f = jiota.astype(jnp.float32)
    gid = i * BLK_I + lax.broadcasted_iota(jnp.int32, (BLK_I, N), 0)
    inf = jnp.float32(jnp.inf)
    dist = jnp.where(jiota == gid, inf, dist)
    big = jnp.float32(2.0 * N)
    cols = []
    for _ in range(K):
        m = jnp.min(dist, axis=1, keepdims=True)
        cand = jnp.where(dist == m, jiota_f, big)
        am = jnp.min(cand, axis=1, keepdims=True)
        cols.append(am.astype(jnp.int32))
        dist = jnp.where(jiota_f == am, inf, dist)

    v = jnp.dot(xf_ref[...], w1t_ref[...], preferred_element_type=jnp.float32,
                precision=jax.lax.Precision.HIGHEST)
    v_blk = jnp.dot(x_blk, w1t_ref[...], preferred_element_type=jnp.float32,
                    precision=jax.lax.Precision.HIGHEST)
    m_blocks = [(jiota == cols[s_]).astype(jnp.bfloat16) for s_ in range(K)]
    m_neg = jnp.concatenate(m_blocks, axis=0)
    v_self = jnp.concatenate([v_blk] * K, axis=0)
    va = v.astype(jnp.bfloat16)
    vn = jnp.dot(m_neg, va, preferred_element_type=jnp.float32)
    h = jnp.maximum((v_self - vn) + b1_ref[...], 0.0)
    out = (jnp.dot(h, w2t_ref[...], preferred_element_type=jnp.float32)
           + b2_ref[...])
    out_ref[0] = out.reshape(K, BLK_I, D)


@functools.partial(jax.jit, static_argnames=("interpret",))
def _run(x, W1, b1, W2, b2, interpret=False):
    xm = x[0]
    xt = xm.T
    out = pl.pallas_call(
        _fused_kernel,
        grid=(GRID,),
        in_specs=[
            pl.BlockSpec((N, D), lambda i: (0, 0)),
            pl.BlockSpec((D, N), lambda i: (0, 0)),
            pl.BlockSpec((D, D), lambda i: (0, 0)),
            pl.BlockSpec((1, D), lambda i: (0, 0)),
            pl.BlockSpec((D, D), lambda i: (0, 0)),
            pl.BlockSpec((1, D), lambda i: (0, 0)),
        ],
        out_specs=pl.BlockSpec((1, K, BLK_I, D), lambda i: (i, 0, 0, 0)),
        out_shape=jax.ShapeDtypeStruct((GRID, K, BLK_I, D), jnp.float32),
        interpret=interpret,
    )(xm, xt, W1.T, b1.reshape(1, D), W2.T, b2.reshape(1, D))
    return out.transpose(0, 2, 1, 3).reshape(1, N, K, D)


def kernel(x, W1, b1, W2, b2, k):
    return _run(x, W1, b1, W2, b2)


# speedup vs baseline: 1.3733x; 1.0228x over previous
import functools

import jax
import jax.numpy as jnp
from jax import lax
from jax.experimental import pallas as pl

N = 1024
D = 64
K = 16
BLK_I = 128
GRID = N // BLK_I
ROWS2 = BLK_I * K


def _fused_kernel(xf_ref, xt_ref, w1t_ref, b1_ref, w2t_ref, b2_ref, out_ref):
    i = pl.program_id(0)
    x_blk = xf_ref[pl.ds(i * BLK_I, BLK_I), :]
    acc = jnp.zeros((BLK_I, N), jnp.float32)
    for g in range(D // 8):
        s = []
        for t in range(8 * g, 8 * g + 8):
            xi = x_blk[:, t:t + 1]
            xj = xt_ref[t:t + 1, :]
            df = xi - xj
            s.append(df * df)
        tree = ((s[0] + s[4]) + (s[2] + s[6])) + ((s[1] + s[5]) + (s[3] + s[7]))
        acc = acc + tree
    dist = jnp.sqrt(acc)
    jiota = lax.broadcasted_iota(jnp.int32, (BLK_I, N), 1)
    jiota_f = jiota.astype(jnp.float32)
    gid = i * BLK_I + lax.broadcasted_iota(jnp.int32, (BLK_I, N), 0)
    inf = jnp.float32(jnp.inf)
    dist = jnp.where(jiota == gid, inf, dist)
    big = jnp.float32(2.0 * N)
    cols = []
    for _ in range(K):
        m = jnp.min(dist, axis=1, keepdims=True)
        cand = jnp.where(dist == m, jiota_f, big)
        am = jnp.min(cand, axis=1, keepdims=True)
        cols.append(am.astype(jnp.int32))
        dist = jnp.where(jiota_f == am, inf, dist)

    v = jnp.dot(xf_ref[...], w1t_ref[...], preferred_element_type=jnp.float32,
                precision=jax.lax.Precision.HIGHEST)
    v_blk = jnp.dot(x_blk, w1t_ref[...], preferred_element_type=jnp.float32,
                    precision=jax.lax.Precision.HIGHEST)
    m_blocks = [(jiota == cols[s_]).astype(jnp.bfloat16) for s_ in range(K)]
    m_neg = jnp.concatenate(m_blocks, axis=0)
    v_self = jnp.concatenate([v_blk] * K, axis=0)
    va = v.astype(jnp.bfloat16)
    vn = jnp.dot(m_neg, va, preferred_element_type=jnp.float32)
    h = jnp.maximum((v_self - vn) + b1_ref[...], 0.0)
    out = (jnp.dot(h, w2t_ref[...], preferred_element_type=jnp.float32)
           + b2_ref[...])
    out_ref[...] = jnp.concatenate(
        [out[s_ * BLK_I:(s_ + 1) * BLK_I, :] for s_ in range(K)], axis=1)


@functools.partial(jax.jit, static_argnames=("interpret",))
def _run(x, W1, b1, W2, b2, interpret=False):
    xm = x[0]
    xt = xm.T
    out = pl.pallas_call(
        _fused_kernel,
        grid=(GRID,),
        in_specs=[
            pl.BlockSpec((N, D), lambda i: (0, 0)),
            pl.BlockSpec((D, N), lambda i: (0, 0)),
            pl.BlockSpec((D, D), lambda i: (0, 0)),
            pl.BlockSpec((1, D), lambda i: (0, 0)),
            pl.BlockSpec((D, D), lambda i: (0, 0)),
            pl.BlockSpec((1, D), lambda i: (0, 0)),
        ],
        out_specs=pl.BlockSpec((BLK_I, K * D), lambda i: (i, 0)),
        out_shape=jax.ShapeDtypeStruct((N, K * D), jnp.float32),
        interpret=interpret,
    )(xm, xt, W1.T, b1.reshape(1, D), W2.T, b2.reshape(1, D))
    return out.reshape(1, N, K, D)


def kernel(x, W1, b1, W2, b2, k):
    return _run(x, W1, b1, W2, b2)


# speedup vs baseline: 1.5681x; 1.1418x over previous
import functools

import jax
import jax.numpy as jnp
from jax import lax
from jax.experimental import pallas as pl

N = 1024
D = 64
K = 16
BLK_I = 256
GRID = N // BLK_I
ROWS2 = BLK_I * K


def _fused_kernel(xf_ref, xt_ref, w1t_ref, b1_ref, w2t_ref, b2_ref, out_ref):
    i = pl.program_id(0)
    x_blk = xf_ref[pl.ds(i * BLK_I, BLK_I), :]
    acc = jnp.zeros((BLK_I, N), jnp.float32)
    for g in range(D // 8):
        s = []
        for t in range(8 * g, 8 * g + 8):
            xi = x_blk[:, t:t + 1]
            xj = xt_ref[t:t + 1, :]
            df = xi - xj
            s.append(df * df)
        tree = ((s[0] + s[4]) + (s[2] + s[6])) + ((s[1] + s[5]) + (s[3] + s[7]))
        acc = acc + tree
    dist = jnp.sqrt(acc)
    jiota = lax.broadcasted_iota(jnp.int32, (BLK_I, N), 1)
    jiota_f = jiota.astype(jnp.float32)
    gid = i * BLK_I + lax.broadcasted_iota(jnp.int32, (BLK_I, N), 0)
    inf = jnp.float32(jnp.inf)
    dist = jnp.where(jiota == gid, inf, dist)
    big = jnp.float32(2.0 * N)
    cols = []
    for _ in range(K):
        m = jnp.min(dist, axis=1, keepdims=True)
        cand = jnp.where(dist == m, jiota_f, big)
        am = jnp.min(cand, axis=1, keepdims=True)
        cols.append(am.astype(jnp.int32))
        dist = jnp.where(jiota_f == am, inf, dist)

    v = jnp.dot(xf_ref[...], w1t_ref[...], preferred_element_type=jnp.float32,
                precision=jax.lax.Precision.HIGHEST)
    v_blk = jnp.dot(x_blk, w1t_ref[...], preferred_element_type=jnp.float32,
                    precision=jax.lax.Precision.HIGHEST)
    m_blocks = [(jiota == cols[s_]).astype(jnp.bfloat16) for s_ in range(K)]
    m_neg = jnp.concatenate(m_blocks, axis=0)
    v_self = jnp.concatenate([v_blk] * K, axis=0)
    va = v.astype(jnp.bfloat16)
    vn = jnp.dot(m_neg, va, preferred_element_type=jnp.float32)
    h = jnp.maximum((v_self - vn) + b1_ref[...], 0.0)
    out = (jnp.dot(h, w2t_ref[...], preferred_element_type=jnp.float32)
           + b2_ref[...])
    out_ref[...] = jnp.concatenate(
        [out[s_ * BLK_I:(s_ + 1) * BLK_I, :] for s_ in range(K)], axis=1)


@functools.partial(jax.jit, static_argnames=("interpret",))
def _run(x, W1, b1, W2, b2, interpret=False):
    xm = x[0]
    xt = xm.T
    out = pl.pallas_call(
        _fused_kernel,
        grid=(GRID,),
        in_specs=[
            pl.BlockSpec((N, D), lambda i: (0, 0)),
            pl.BlockSpec((D, N), lambda i: (0, 0)),
            pl.BlockSpec((D, D), lambda i: (0, 0)),
            pl.BlockSpec((1, D), lambda i: (0, 0)),
            pl.BlockSpec((D, D), lambda i: (0, 0)),
            pl.BlockSpec((1, D), lambda i: (0, 0)),
        ],
        out_specs=pl.BlockSpec((BLK_I, K * D), lambda i: (i, 0)),
        out_shape=jax.ShapeDtypeStruct((N, K * D), jnp.float32),
        interpret=interpret,
    )(xm, xt, W1.T, b1.reshape(1, D), W2.T, b2.reshape(1, D))
    return out.reshape(1, N, K, D)


def kernel(x, W1, b1, W2, b2, k):
    return _run(x, W1, b1, W2, b2)


# speedup vs baseline: 1.5985x; 1.0194x over previous
import functools

import jax
import jax.numpy as jnp
from jax import lax
from jax.experimental import pallas as pl

N = 1024
D = 64
K = 16
BLK_I = 512
GRID = N // BLK_I
ROWS2 = BLK_I * K


def _fused_kernel(xf_ref, xt_ref, w1t_ref, b1_ref, w2t_ref, b2_ref, out_ref):
    i = pl.program_id(0)
    x_blk = xf_ref[pl.ds(i * BLK_I, BLK_I), :]
    acc = jnp.zeros((BLK_I, N), jnp.float32)
    for g in range(D // 8):
        s = []
        for t in range(8 * g, 8 * g + 8):
            xi = x_blk[:, t:t + 1]
            xj = xt_ref[t:t + 1, :]
            df = xi - xj
            s.append(df * df)
        tree = ((s[0] + s[4]) + (s[2] + s[6])) + ((s[1] + s[5]) + (s[3] + s[7]))
        acc = acc + tree
    dist = jnp.sqrt(acc)
    jiota = lax.broadcasted_iota(jnp.int32, (BLK_I, N), 1)
    jiota_f = jiota.astype(jnp.float32)
    gid = i * BLK_I + lax.broadcasted_iota(jnp.int32, (BLK_I, N), 0)
    inf = jnp.float32(jnp.inf)
    dist = jnp.where(jiota == gid, inf, dist)
    big = jnp.float32(2.0 * N)
    cols = []
    for _ in range(K):
        m = jnp.min(dist, axis=1, keepdims=True)
        cand = jnp.where(dist == m, jiota_f, big)
        am = jnp.min(cand, axis=1, keepdims=True)
        cols.append(am.astype(jnp.int32))
        dist = jnp.where(jiota_f == am, inf, dist)

    v = jnp.dot(xf_ref[...], w1t_ref[...], preferred_element_type=jnp.float32,
                precision=jax.lax.Precision.HIGHEST)
    v_blk = jnp.dot(x_blk, w1t_ref[...], preferred_element_type=jnp.float32,
                    precision=jax.lax.Precision.HIGHEST)
    m_blocks = [(jiota == cols[s_]).astype(jnp.bfloat16) for s_ in range(K)]
    m_neg = jnp.concatenate(m_blocks, axis=0)
    v_self = jnp.concatenate([v_blk] * K, axis=0)
    va = v.astype(jnp.bfloat16)
    vn = jnp.dot(m_neg, va, preferred_element_type=jnp.float32)
    h = jnp.maximum((v_self - vn) + b1_ref[...], 0.0)
    out = (jnp.dot(h, w2t_ref[...], preferred_element_type=jnp.float32)
           + b2_ref[...])
    out_ref[...] = jnp.concatenate(
        [out[s_ * BLK_I:(s_ + 1) * BLK_I, :] for s_ in range(K)], axis=1)


@functools.partial(jax.jit, static_argnames=("interpret",))
def _run(x, W1, b1, W2, b2, interpret=False):
    xm = x[0]
    xt = xm.T
    out = pl.pallas_call(
        _fused_kernel,
        grid=(GRID,),
        in_specs=[
            pl.BlockSpec((N, D), lambda i: (0, 0)),
            pl.BlockSpec((D, N), lambda i: (0, 0)),
            pl.BlockSpec((D, D), lambda i: (0, 0)),
            pl.BlockSpec((1, D), lambda i: (0, 0)),
            pl.BlockSpec((D, D), lambda i: (0, 0)),
            pl.BlockSpec((1, D), lambda i: (0, 0)),
        ],
        out_specs=pl.BlockSpec((BLK_I, K * D), lambda i: (i, 0)),
        out_shape=jax.ShapeDtypeStruct((N, K * D), jnp.float32),
        interpret=interpret,
    )(xm, xt, W1.T, b1.reshape(1, D), W2.T, b2.reshape(1, D))
    return out.reshape(1, N, K, D)


def kernel(x, W1, b1, W2, b2, k):
    return _run(x, W1, b1, W2, b2)
